# baseline (speedup 1.0000x reference)
import functools

import jax
import jax.numpy as jnp
from jax import lax
from jax.experimental import pallas as pl
from jax.experimental.pallas import tpu as pltpu
from jax.experimental.pallas import tpu_sc as plsc

EPS = 1e-5
C = 512
NROWS = 8 * 16 * 16 * 16
NV = C // 16
TILE_ROWS = NROWS // 32
CHUNK = 32
NCHUNK = TILE_ROWS // CHUNK
TPAD = 16

_MESH = plsc.VectorSubcoreMesh(core_axis_name="c", subcore_axis_name="s")


def _factor(m2):
    i = lax.bitcast_convert_type(m2, jnp.int32)
    i = jnp.int32(0x5F3759DF) - (i >> 1)
    y = lax.bitcast_convert_type(i, jnp.float32)
    y = y * (1.5 - 0.5 * m2 * y * y)
    y = y * (1.5 - 0.5 * m2 * y * y)
    return (m2 * y - 1.0) * jnp.minimum(y, 1.0)


def _loop_split(lo, hi, fn, parts, unroll):
    bounds = [lo + (hi - lo) * p // parts for p in range(parts + 1)]
    for p in range(parts):

        @plsc.parallel_loop(0, CHUNK, unroll=unroll)
        def part(r, _lo=bounds[p], _hi=bounds[p + 1]):
            fn(r, _lo, _hi)

        del part


def _sum3(vm2, vm1, v0, vp1, vp2, is_p0, is_p1):
    a, b, d, e, f = vm2 * vm2, vm1 * vm1, v0 * v0, vp1 * vp1, vp2 * vp2
    de = d + e
    return jnp.where(is_p0, de + f, jnp.where(is_p1, b + de, a + b + d))


def _compute(bufs, obs, tb, par):
    iota = lax.iota(jnp.int32, 16)
    g1_m = []
    for a in range(2, 14):
        p = (16 * a + iota - 32) % 3
        g1_m.append((p == 0, p == 1))
    g2a_m = []
    g2b_m = []
    for a in range(14, 32):
        q = (16 * a + iota - 224) % 3
        g2a_m.append((q == 0, q == 1))
        r9 = (16 * a + iota - 224) % 9
        g2b_m.append((r9 < 3, (r9 >= 3) & (r9 < 6)))

    def g2a(r, a_lo, a_hi):
        for a in range(a_lo, a_hi):
            k = a - 14
            c0 = 16 * a
            v0 = bufs[par, r, pl.ds(c0, 16)]
            vm2 = bufs[par, r, pl.ds(c0 - 2, 16)]
            vm1 = bufs[par, r, pl.ds(c0 - 1, 16)]
            if a == 31:
                vp1 = bufs[par, r, pl.ds(r * 0 + (c0 + 1), 16)]
                vp2 = bufs[par, r, pl.ds(r * 0 + (c0 + 2), 16)]
            else:
                vp1 = bufs[par, r, pl.ds(c0 + 1, 16)]
                vp2 = bufs[par, r, pl.ds(c0 + 2, 16)]
            tb[r, pl.ds(TPAD + 16 * k, 16)] = _sum3(
                vm2, vm1, v0, vp1, vp2, g2a_m[k][0], g2a_m[k][1]
            )

    def g1(r, a_lo, a_hi):
        for a in range(a_lo, a_hi):
            k = a - 2
            c0 = 16 * a
            v0 = bufs[par, r, pl.ds(c0, 16)]
            vm2 = bufs[par, r, pl.ds(c0 - 2, 16)]
            vm1 = bufs[par, r, pl.ds(c0 - 1, 16)]
            vp1 = bufs[par, r, pl.ds(c0 + 1, 16)]
            vp2 = bufs[par, r, pl.ds(c0 + 2, 16)]
            m2 = _sum3(vm2, vm1, v0, vp1, vp2, g1_m[k][0], g1_m[k][1]) + EPS
            obs[par, r, pl.ds(c0, 16)] = v0 * _factor(m2)

    def g0(r, a_lo, a_hi):
        for a in range(a_lo, a_hi):
            col = pl.ds(16 * a, 16)
            v = bufs[par, r, col]
            obs[par, r, col] = v * _factor(v * v + EPS)

    def g2b(r, a_lo, a_hi):
        for a in range(a_lo, a_hi):
            k = a - 14
            c0 = 16 * a
            tc = TPAD + 16 * k
            t0 = tb[r, pl.ds(tc, 16)]
            tm3 = tb[r, pl.ds(tc - 3, 16)]
            tm6 = tb[r, pl.ds(tc - 6, 16)]
            tp3 = tb[r, pl.ds(tc + 3, 16)]
            tp6 = tb[r, pl.ds(tc + 6, 16)]
            u = t0 + tp3
            m2 = (
                jnp.where(
                    g2b_m[k][0],
                    u + tp6,
                    jnp.where(g2b_m[k][1], tm3 + u, tm6 + tm3 + t0),
                )
                + EPS
            )
            col = pl.ds(c0, 16)
            obs[par, r, col] = bufs[par, r, col] * _factor(m2)

    _loop_split(14, 32, g2a, parts=18, unroll=1)
    _loop_split(2, 14, g1, parts=12, unroll=1)

    @plsc.parallel_loop(0, CHUNK, unroll=2)
    def g0loop(r):
        g0(r, 0, 2)

    del g0loop
    _loop_split(14, 32, g2b, parts=18, unroll=1)


def _body(x_hbm, o_hbm, bufs, obs, tb, isem, osem):
    w = lax.axis_index("s") * 2 + lax.axis_index("c")
    row0 = w * TILE_ROWS

    def in_cp(c, par):
        return pltpu.make_async_copy(
            x_hbm.at[pl.ds(row0 + c * CHUNK, CHUNK)],
            bufs.at[par, pl.ds(0, CHUNK)],
            isem.at[par],
        )

    def out_cp(c, par):
        return pltpu.make_async_copy(
            obs.at[par],
            o_hbm.at[pl.ds(row0 + c * CHUNK, CHUNK)],
            osem.at[par],
        )

    in_cp(0, 0).start()
    in_cp(1, 1).start()

    def step(c, carry):
        par = c % 2
        in_cp(c, par).wait()

        @pl.when(c >= 2)
        def _():
            out_cp(c - 2, par).wait()

        _compute(bufs, obs, tb, par)
        out_cp(c, par).start()

        @pl.when(c + 2 < NCHUNK)
        def _():
            in_cp(c + 2, par).start()

        return carry

    lax.fori_loop(0, NCHUNK, step, 0)
    out_cp(NCHUNK - 2, 0).wait()
    out_cp(NCHUNK - 1, 1).wait()


_sc_call = functools.partial(
    pl.kernel,
    out_type=jax.ShapeDtypeStruct((NROWS, C), jnp.float32),
    mesh=_MESH,
    scratch_types=[
        pltpu.VMEM((2, CHUNK + 1, C), jnp.float32),
        pltpu.VMEM((2, CHUNK, C), jnp.float32),
        pltpu.VMEM((CHUNK, 16 * 18 + 2 * TPAD), jnp.float32),
        pltpu.SemaphoreType.DMA((2,)),
        pltpu.SemaphoreType.DMA((2,)),
    ],
)(_body)


def kernel(x):
    xt = jnp.transpose(x, (0, 2, 3, 4, 1)).reshape(NROWS, C)
    out = _sc_call(xt)
    return jnp.transpose(out.reshape(8, 16, 16, 16, C), (0, 4, 1, 2, 3))


# speedup vs baseline: 1.0921x; 1.0921x over previous
import functools

import jax
import jax.numpy as jnp
from jax import lax
from jax.experimental import pallas as pl
from jax.experimental.pallas import tpu as pltpu
from jax.experimental.pallas import tpu_sc as plsc

EPS = 1e-5
C = 512
NROWS = 8 * 16 * 16 * 16
NV = C // 16
TILE_ROWS = NROWS // 32
CHUNK = 32
NCHUNK = TILE_ROWS // CHUNK
TPAD = 16

_MESH = plsc.VectorSubcoreMesh(core_axis_name="c", subcore_axis_name="s")


def _factor(m2):
    i = lax.bitcast_convert_type(m2, jnp.int32)
    i = jnp.int32(0x5F3759DF) - (i >> 1)
    y = lax.bitcast_convert_type(i, jnp.float32)
    y = y * (1.5 - 0.5 * m2 * y * y)
    return (m2 * y - 1.0) * jnp.minimum(y, 1.0)


def _loop_split(lo, hi, fn, parts, unroll):
    bounds = [lo + (hi - lo) * p // parts for p in range(parts + 1)]
    for p in range(parts):

        @plsc.parallel_loop(0, CHUNK, unroll=unroll)
        def part(r, _lo=bounds[p], _hi=bounds[p + 1]):
            fn(r, _lo, _hi)

        del part


def _sum3(vm2, vm1, v0, vp1, vp2, is_p0, is_p1):
    a, b, d, e, f = vm2 * vm2, vm1 * vm1, v0 * v0, vp1 * vp1, vp2 * vp2
    de = d + e
    return jnp.where(is_p0, de + f, jnp.where(is_p1, b + de, a + b + d))


def _compute(bufs, obs, tb, par):
    iota = lax.iota(jnp.int32, 16)
    g1_m = []
    for a in range(2, 14):
        p = (16 * a + iota - 32) % 3
        g1_m.append((p == 0, p == 1))
    g2a_m = []
    g2b_m = []
    for a in range(14, 32):
        q = (16 * a + iota - 224) % 3
        g2a_m.append((q == 0, q == 1))
        r9 = (16 * a + iota - 224) % 9
        g2b_m.append((r9 < 3, (r9 >= 3) & (r9 < 6)))

    def g2a(r, a_lo, a_hi):
        for a in range(a_lo, a_hi):
            k = a - 14
            c0 = 16 * a
            v0 = bufs[par, r, pl.ds(c0, 16)]
            vm2 = bufs[par, r, pl.ds(c0 - 2, 16)]
            vm1 = bufs[par, r, pl.ds(c0 - 1, 16)]
            if a == 31:
                vp1 = bufs[par, r, pl.ds(r * 0 + (c0 + 1), 16)]
                vp2 = bufs[par, r, pl.ds(r * 0 + (c0 + 2), 16)]
            else:
                vp1 = bufs[par, r, pl.ds(c0 + 1, 16)]
                vp2 = bufs[par, r, pl.ds(c0 + 2, 16)]
            tb[r, pl.ds(TPAD + 16 * k, 16)] = _sum3(
                vm2, vm1, v0, vp1, vp2, g2a_m[k][0], g2a_m[k][1]
            )

    def g1(r, a_lo, a_hi):
        for a in range(a_lo, a_hi):
            k = a - 2
            c0 = 16 * a
            v0 = bufs[par, r, pl.ds(c0, 16)]
            vm2 = bufs[par, r, pl.ds(c0 - 2, 16)]
            vm1 = bufs[par, r, pl.ds(c0 - 1, 16)]
            vp1 = bufs[par, r, pl.ds(c0 + 1, 16)]
            vp2 = bufs[par, r, pl.ds(c0 + 2, 16)]
            m2 = _sum3(vm2, vm1, v0, vp1, vp2, g1_m[k][0], g1_m[k][1]) + EPS
            obs[par, r, pl.ds(c0, 16)] = v0 * _factor(m2)

    def g0(r, a_lo, a_hi):
        for a in range(a_lo, a_hi):
            col = pl.ds(16 * a, 16)
            v = bufs[par, r, col]
            obs[par, r, col] = v * _factor(v * v + EPS)

    def g2b(r, a_lo, a_hi):
        for a in range(a_lo, a_hi):
            k = a - 14
            c0 = 16 * a
            tc = TPAD + 16 * k
            t0 = tb[r, pl.ds(tc, 16)]
            tm3 = tb[r, pl.ds(tc - 3, 16)]
            tm6 = tb[r, pl.ds(tc - 6, 16)]
            tp3 = tb[r, pl.ds(tc + 3, 16)]
            tp6 = tb[r, pl.ds(tc + 6, 16)]
            u = t0 + tp3
            m2 = (
                jnp.where(
                    g2b_m[k][0],
                    u + tp6,
                    jnp.where(g2b_m[k][1], tm3 + u, tm6 + tm3 + t0),
                )
                + EPS
            )
            col = pl.ds(c0, 16)
            obs[par, r, col] = bufs[par, r, col] * _factor(m2)

    _loop_split(14, 32, g2a, parts=18, unroll=1)
    _loop_split(2, 14, g1, parts=12, unroll=1)

    @plsc.parallel_loop(0, CHUNK, unroll=2)
    def g0loop(r):
        g0(r, 0, 2)

    del g0loop
    _loop_split(14, 32, g2b, parts=18, unroll=1)


def _body(x_hbm, o_hbm, bufs, obs, tb, isem, osem):
    w = lax.axis_index("s") * 2 + lax.axis_index("c")
    row0 = w * TILE_ROWS

    def in_cp(c, par):
        return pltpu.make_async_copy(
            x_hbm.at[pl.ds(row0 + c * CHUNK, CHUNK)],
            bufs.at[par, pl.ds(0, CHUNK)],
            isem.at[par],
        )

    def out_cp(c, par):
        return pltpu.make_async_copy(
            obs.at[par],
            o_hbm.at[pl.ds(row0 + c * CHUNK, CHUNK)],
            osem.at[par],
        )

    in_cp(0, 0).start()
    in_cp(1, 1).start()

    def step(c, carry):
        par = c % 2
        in_cp(c, par).wait()

        @pl.when(c >= 2)
        def _():
            out_cp(c - 2, par).wait()

        _compute(bufs, obs, tb, par)
        out_cp(c, par).start()

        @pl.when(c + 2 < NCHUNK)
        def _():
            in_cp(c + 2, par).start()

        return carry

    lax.fori_loop(0, NCHUNK, step, 0)
    out_cp(NCHUNK - 2, 0).wait()
    out_cp(NCHUNK - 1, 1).wait()


_sc_call = functools.partial(
    pl.kernel,
    out_type=jax.ShapeDtypeStruct((NROWS, C), jnp.float32),
    mesh=_MESH,
    scratch_types=[
        pltpu.VMEM((2, CHUNK + 1, C), jnp.float32),
        pltpu.VMEM((2, CHUNK, C), jnp.float32),
        pltpu.VMEM((CHUNK, 16 * 18 + 2 * TPAD), jnp.float32),
        pltpu.SemaphoreType.DMA((2,)),
        pltpu.SemaphoreType.DMA((2,)),
    ],
)(_body)


def kernel(x):
    xt = jnp.transpose(x, (0, 2, 3, 4, 1)).reshape(NROWS, C)
    out = _sc_call(xt)
    return jnp.transpose(out.reshape(8, 16, 16, 16, C), (0, 4, 1, 2, 3))
